# baseline (speedup 1.0000x reference)
import functools

import jax
import jax.numpy as jnp
from jax import lax
from jax.experimental import pallas as pl
from jax.experimental.pallas import tpu as pltpu
from jax.experimental.pallas import tpu_sc as plsc

MASK_RATIO_ = 0.75
R, C = 64, 128
KTILE = 2048
SC_CHUNK = 16


def _xor_shuffle(x, d):
    if d < C:
        bit = jax.lax.broadcasted_iota(jnp.int32, (R, C), 1) & d
        return jnp.where(bit != 0, pltpu.roll(x, d, 1), pltpu.roll(x, C - d, 1))
    s = d // C
    bit = jax.lax.broadcasted_iota(jnp.int32, (R, C), 0) & s
    return jnp.where(bit != 0, pltpu.roll(x, s, 0), pltpu.roll(x, R - s, 0))


def _sort_kernel(noise_ref, shuf_ref, *, L):
    ir = jax.lax.broadcasted_iota(jnp.int32, (R, C), 0)
    ic = jax.lax.broadcasted_iota(jnp.int32, (R, C), 1)
    idx = ir * C + ic
    m = (noise_ref[0, :, :] * float(1 << 23)).astype(jnp.int32)

    def bit_of(v):
        if v < C:
            return (ic & v) != 0
        return (ir & (v // C)) != 0

    k = 2
    while k <= L:
        d = k // 2
        while d >= 1:
            pm = _xor_shuffle(m, d)
            pidx = _xor_shuffle(idx, d)
            p_lt = (pm < m) | ((pm == m) & (pidx < idx))
            want_min = bit_of(k) == bit_of(d)
            take = p_lt == want_min
            m = jnp.where(take, pm, m)
            idx = jnp.where(take, pidx, idx)
            d //= 2
        k *= 2
    shuf_ref[0, :, :] = idx


def _restore_kernel(shrow_ref, shcol_ref, rest_ref, mask_ref, *, L, len_keep):
    sh_row = shrow_ref[0, :, :]
    sh_col = shcol_ref[0, :, :]
    ihi = jax.lax.broadcasted_iota(jnp.int32, (R, 1), 0)
    ilo = jax.lax.broadcasted_iota(jnp.int32, (1, C), 1)
    a = ((sh_row >> 7) == ihi).astype(jnp.float32)
    kcol = jax.lax.broadcasted_iota(jnp.int32, (L, 1), 0)
    b = jnp.where((sh_col & (C - 1)) == ilo, kcol, 0).astype(jnp.float32)
    mres = jnp.dot(a, b, precision=jax.lax.Precision.HIGHEST)
    rest = mres.astype(jnp.int32)
    rest_ref[0, :, :] = rest
    mask_ref[0, :, :] = jnp.where(mres >= float(len_keep), 1.0, 0.0)


def _bcast_kernel(col_ref, out_ref, *, E, rows):
    out_ref[0, :, :] = jnp.broadcast_to(col_ref[0, :, :], (rows, E))


def _expand16_kernel(col_ref, keep_ref, mask_ref, *, L, len_keep):
    col = col_ref[0, :, :]
    keep_ref[0, :, :] = jnp.broadcast_to(col[:len_keep, :], (len_keep, 16))
    mask_ref[0, :, :] = jnp.broadcast_to(col[len_keep:, :], (L - len_keep, 16))


def _make_sc_writer(nrows, E):
    info = plsc.get_sparse_core_info()
    NC, NS = info.num_cores, info.num_subcores
    NW = NC * NS
    rows_w = nrows // NW
    nch = rows_w // SC_CHUNK
    assert rows_w % SC_CHUNK == 0 and nch % 2 == 0
    groups = E // 16
    mesh = plsc.VectorSubcoreMesh(core_axis_name="c", subcore_axis_name="s")

    @functools.partial(
        pl.kernel, mesh=mesh,
        out_type=jax.ShapeDtypeStruct((nrows * E,), jnp.int32),
        scratch_types=[
            pltpu.VMEM((rows_w * 16,), jnp.int32),
            pltpu.VMEM((SC_CHUNK * E,), jnp.int32),
            pltpu.VMEM((SC_CHUNK * E,), jnp.int32),
            pltpu.SemaphoreType.DMA,
            pltpu.SemaphoreType.DMA,
        ],
    )
    def writer(vals_hbm, out_hbm, vals_v, buf0, buf1, sem0, sem1):
        wid = lax.axis_index("s") * NC + lax.axis_index("c")
        base = wid * rows_w
        pltpu.sync_copy(vals_hbm.at[pl.ds(base * 16, rows_w * 16)], vals_v)
        bufs, sems = (buf0, buf1), (sem0, sem1)

        def fill_fire(c, b2):
            buf = bufs[b2]
            for r in range(SC_CHUNK):
                v = vals_v[pl.ds((c * SC_CHUNK + r) * 16, 16)]
                for j in range(groups):
                    buf[pl.ds(r * E + j * 16, 16)] = v
            pltpu.make_async_copy(
                buf,
                out_hbm.at[pl.ds((base + c * SC_CHUNK) * E, SC_CHUNK * E)],
                sems[b2]).start()

        def wait_b(b2):
            pltpu.make_async_copy(
                bufs[b2],
                out_hbm.at[pl.ds(base * E, SC_CHUNK * E)],
                sems[b2]).wait()

        fill_fire(0, 0)
        fill_fire(1, 1)

        def body(t, _):
            for b2 in range(2):
                c = t * 2 + b2
                wait_b(b2)
                fill_fire(c, b2)
            return _

        lax.fori_loop(1, nch // 2, body, None)
        wait_b(0)
        wait_b(1)

    return writer


def kernel(x):
    B, L, E = x.shape
    len_keep = int(L * (1.0 - MASK_RATIO_))
    noise = jax.random.uniform(
        jax.random.fold_in(jax.random.key(0), 1), (B, L), dtype=jnp.float32)
    noise_g = noise.reshape(B, R, C)

    shuf = pl.pallas_call(
        functools.partial(_sort_kernel, L=L),
        grid=(B,),
        in_specs=[pl.BlockSpec((1, R, C), lambda b: (b, 0, 0))],
        out_specs=pl.BlockSpec((1, R, C), lambda b: (b, 0, 0)),
        out_shape=jax.ShapeDtypeStruct((B, R, C), jnp.int32),
    )(noise_g)

    sh_row = shuf.reshape(B, 1, L)
    sh_col = shuf.reshape(B, L, 1)

    rest, mask = pl.pallas_call(
        functools.partial(_restore_kernel, L=L, len_keep=len_keep),
        grid=(B,),
        in_specs=[
            pl.BlockSpec((1, 1, L), lambda b: (b, 0, 0)),
            pl.BlockSpec((1, L, 1), lambda b: (b, 0, 0)),
        ],
        out_specs=[
            pl.BlockSpec((1, R, C), lambda b: (b, 0, 0)),
            pl.BlockSpec((1, R, C), lambda b: (b, 0, 0)),
        ],
        out_shape=[
            jax.ShapeDtypeStruct((B, R, C), jnp.int32),
            jax.ShapeDtypeStruct((B, R, C), jnp.float32),
        ],
    )(sh_row, sh_col)

    def bcast(nrows, off):
        rows = min(KTILE, nrows)
        return pl.pallas_call(
            functools.partial(_bcast_kernel, E=E, rows=rows),
            grid=(B, nrows // rows),
            in_specs=[pl.BlockSpec((1, rows, 1), lambda b, k: (b, k + off, 0))],
            out_specs=pl.BlockSpec((1, rows, E), lambda b, k: (b, k, 0)),
            out_shape=jax.ShapeDtypeStruct((B, nrows, E), jnp.int32),
        )(sh_col)

    keep16, mask16 = pl.pallas_call(
        functools.partial(_expand16_kernel, L=L, len_keep=len_keep),
        grid=(B,),
        in_specs=[pl.BlockSpec((1, L, 1), lambda b: (b, 0, 0))],
        out_specs=[
            pl.BlockSpec((1, len_keep, 16), lambda b: (b, 0, 0)),
            pl.BlockSpec((1, L - len_keep, 16), lambda b: (b, 0, 0)),
        ],
        out_shape=[
            jax.ShapeDtypeStruct((B, len_keep, 16), jnp.int32),
            jax.ShapeDtypeStruct((B, L - len_keep, 16), jnp.int32),
        ],
    )(sh_col)

    ids_keep = _make_sc_writer(B * len_keep, E)(
        keep16.reshape(B * len_keep * 16)).reshape(B, len_keep, E)
    ids_mask = bcast(L - len_keep, len_keep // KTILE)

    return (mask.reshape(B, L), ids_keep, rest.reshape(B, L), ids_mask)
